# baseline (speedup 1.0000x reference)
import jax
import jax.numpy as jnp
from jax import lax
from jax.experimental import pallas as pl
from jax.experimental.pallas import tpu as pltpu
from jax.experimental.pallas import tpu_sc as plsc

_BC = 16384


def _ent_add_kernel(e0, e1, out_e):
    out_e[...] = e0[...] + e1[...]


def _ent_add(e0t, e1t):
    d_ent, n_ent = e0t.shape
    ent_spec = pl.BlockSpec((d_ent, _BC), lambda i: (0, i))
    return pl.pallas_call(
        _ent_add_kernel,
        grid=(pl.cdiv(n_ent, _BC),),
        in_specs=[ent_spec, ent_spec],
        out_specs=ent_spec,
        out_shape=jax.ShapeDtypeStruct((d_ent, n_ent), e0t.dtype),
        compiler_params=pltpu.CompilerParams(
            dimension_semantics=("arbitrary",),
        ),
    )(e0t, e1t)


def _rel_sc_body(r0_hbm, r1_hbm, out_hbm, a, b, c, sem0, sem1):
    w = lax.axis_index("s") * 2 + lax.axis_index("c")
    rows = pl.ds(2 * w, 2)
    cp0 = pltpu.make_async_copy(r0_hbm.at[rows, :], a, sem0)
    cp1 = pltpu.make_async_copy(r1_hbm.at[rows, :], b, sem1)
    cp0.start()
    cp1.start()
    cp0.wait()
    cp1.wait()
    n = a.shape[1]
    offs = [16 * j for j in range(n // 16)]
    if n % 16:
        offs.append(n - 16)
    for i in range(a.shape[0]):
        for off in offs:
            sl = pl.ds(off, 16)
            c[i, sl] = a[i, sl] + b[i, sl]
    pltpu.sync_copy(c, out_hbm.at[rows, :])


def _rel_add(r0t, r1t):
    d_rel, n_rel = r0t.shape
    rows_per_w = d_rel // 32
    k = pl.kernel(
        _rel_sc_body,
        out_type=jax.ShapeDtypeStruct((d_rel, n_rel), r0t.dtype),
        mesh=plsc.VectorSubcoreMesh(core_axis_name="c", subcore_axis_name="s"),
        scratch_types=[
            pltpu.VMEM((rows_per_w, n_rel), r0t.dtype),
            pltpu.VMEM((rows_per_w, n_rel), r0t.dtype),
            pltpu.VMEM((rows_per_w, n_rel), r0t.dtype),
            pltpu.SemaphoreType.DMA,
            pltpu.SemaphoreType.DMA,
        ],
    )
    return k(r0t, r1t)


def kernel(inputs, ent_embeds_0, rel_embeds_0, ent_embeds_1, rel_embeds_1):
    out_et = _ent_add(ent_embeds_0.T, ent_embeds_1.T)
    out_rt = _rel_add(rel_embeds_0.T, rel_embeds_1.T)
    return (out_et.T, out_rt.T)


# speedup vs baseline: 1.6066x; 1.6066x over previous
import jax
import jax.numpy as jnp
from jax.experimental import pallas as pl
from jax.experimental.pallas import tpu as pltpu

_BC = 16384


def _add_kernel(e0, e1, r0, r1, out_e, out_r):
    out_e[...] = e0[...] + e1[...]

    @pl.when(pl.program_id(0) == 0)
    def _():
        out_r[...] = r0[...] + r1[...]


def kernel(inputs, ent_embeds_0, rel_embeds_0, ent_embeds_1, rel_embeds_1):
    n_ent, d_ent = ent_embeds_0.shape
    n_rel, d_rel = rel_embeds_0.shape
    e0t, e1t = ent_embeds_0.T, ent_embeds_1.T
    r0t, r1t = rel_embeds_0.T, rel_embeds_1.T
    grid = (pl.cdiv(n_ent, _BC),)
    ent_spec = pl.BlockSpec((d_ent, _BC), lambda i: (0, i))
    rel_spec = pl.BlockSpec((d_rel, n_rel), lambda i: (0, 0))
    out_et, out_rt = pl.pallas_call(
        _add_kernel,
        grid=grid,
        in_specs=[ent_spec, ent_spec, rel_spec, rel_spec],
        out_specs=[ent_spec, rel_spec],
        out_shape=[
            jax.ShapeDtypeStruct((d_ent, n_ent), ent_embeds_0.dtype),
            jax.ShapeDtypeStruct((d_rel, n_rel), rel_embeds_0.dtype),
        ],
        compiler_params=pltpu.CompilerParams(
            dimension_semantics=("parallel",),
        ),
    )(e0t, e1t, r0t, r1t)
    return (out_et.T, out_rt.T)
